# baseline (speedup 1.0000x reference)
import jax
import jax.numpy as jnp
from jax.experimental import pallas as pl

_EPS = 1e-5
_R = 16
_B = 1000


def _accum(o_ref, part):
    @pl.when(pl.program_id(0) == 0)
    def _():
        o_ref[...] = part

    @pl.when(pl.program_id(0) != 0)
    def _():
        o_ref[...] += part


def _pack(xb, d):
    m = xb.shape[0]
    L = m // 4
    return jnp.concatenate([xb[0:L], xb[L:2 * L], xb[2 * L:3 * L], xb[3 * L:4 * L]],
                           axis=1)


def _colstats(h):
    s = jnp.sum(h, axis=0, keepdims=True)
    sq = jnp.sum(h * h, axis=0, keepdims=True)
    return jnp.concatenate([s, sq], axis=0)


def _p1_kernel(x_ref, o_ref):
    b, r, d = x_ref.shape
    xb = x_ref[...].reshape(b * r, d)
    _accum(o_ref, _colstats(_pack(xb, d)))


def _p2_kernel(x_ref, w0_ref, b0_ref, o_ref):
    b, r, d = x_ref.shape
    xq = _pack(x_ref[...].reshape(b * r, d), d)
    h = jnp.dot(xq, w0_ref[...], preferred_element_type=jnp.float32)
    h = jnp.maximum(h + b0_ref[...], 0.0)
    _accum(o_ref, _colstats(h))


def _p3_kernel(x_ref, w0_ref, b0_ref, w1_ref, b1_ref, o_ref):
    b, r, d = x_ref.shape
    xq = _pack(x_ref[...].reshape(b * r, d), d)
    h = jnp.dot(xq, w0_ref[...], preferred_element_type=jnp.float32)
    h = jnp.maximum(h + b0_ref[...], 0.0)
    h = jnp.dot(h, w1_ref[...], preferred_element_type=jnp.float32)
    h = jnp.maximum(h + b1_ref[...], 0.0)
    _accum(o_ref, _colstats(h))


def _p4_kernel(x_ref, w0_ref, b0_ref, w1_ref, b1_ref, w2_ref, b2_ref, o_ref):
    b, r, d = x_ref.shape
    xq = _pack(x_ref[...].reshape(b * r, d), d)
    h = jnp.dot(xq, w0_ref[...], preferred_element_type=jnp.float32)
    h = jnp.maximum(h + b0_ref[...], 0.0)
    h = jnp.dot(h, w1_ref[...], preferred_element_type=jnp.float32)
    h = jnp.maximum(h + b1_ref[...], 0.0)
    h = jnp.dot(h, w2_ref[...], preferred_element_type=jnp.float32)
    h = jnp.maximum(h + b2_ref[...], 0.0)
    out = jnp.concatenate([h[:, 0:32], h[:, 32:64], h[:, 64:96], h[:, 96:128]],
                          axis=0)
    o_ref[...] = out.reshape(b, r, 32)


def _fold_packed(stats_packed, n, gamma, beta, W, b, groups):
    d = W.shape[0]
    stats = jnp.sum(stats_packed.reshape(2, groups, d), axis=1)
    mean = stats[0] / n
    var = stats[1] / n - mean * mean
    inv = gamma * jax.lax.rsqrt(var + _EPS)
    Wf = W * inv[:, None]
    bf = b + (beta - mean * inv) @ W
    Wbig = jnp.kron(jnp.eye(4, dtype=jnp.float32), Wf)
    bbig = jnp.tile(bf, 4)[None, :]
    return Wbig, bbig


def _full(shape):
    return pl.BlockSpec(shape, lambda i: tuple(0 for _ in shape))


def kernel(x, bn_g0, bn_b0, W0, b0, bn_g1, bn_b1, W1, b1, bn_g2, bn_b2,
           W2, b2):
    n, d_in = x.shape
    dim = W0.shape[1]
    x3 = x.reshape(n // _R, _R, d_in)
    grid = (n // (_R * _B),)
    x_spec = pl.BlockSpec((_B, _R, d_in), lambda i: (i, 0, 0))

    stats0 = pl.pallas_call(
        _p1_kernel,
        grid=grid,
        in_specs=[x_spec],
        out_specs=_full((2, 4 * d_in)),
        out_shape=jax.ShapeDtypeStruct((2, 4 * d_in), jnp.float32),
    )(x3)
    W0f, b0f = _fold_packed(stats0, n, bn_g0, bn_b0, W0, b0, 4)

    stats1 = pl.pallas_call(
        _p2_kernel,
        grid=grid,
        in_specs=[x_spec, _full(W0f.shape), _full(b0f.shape)],
        out_specs=_full((2, 4 * dim)),
        out_shape=jax.ShapeDtypeStruct((2, 4 * dim), jnp.float32),
    )(x3, W0f, b0f)
    W1f, b1f = _fold_packed(stats1, n, bn_g1, bn_b1, W1, b1, 4)

    stats2 = pl.pallas_call(
        _p3_kernel,
        grid=grid,
        in_specs=[x_spec, _full(W0f.shape), _full(b0f.shape),
                  _full(W1f.shape), _full(b1f.shape)],
        out_specs=_full((2, 4 * dim)),
        out_shape=jax.ShapeDtypeStruct((2, 4 * dim), jnp.float32),
    )(x3, W0f, b0f, W1f, b1f)
    W2f, b2f = _fold_packed(stats2, n, bn_g2, bn_b2, W2, b2, 4)

    out3 = pl.pallas_call(
        _p4_kernel,
        grid=grid,
        in_specs=[x_spec, _full(W0f.shape), _full(b0f.shape),
                  _full(W1f.shape), _full(b1f.shape),
                  _full(W2f.shape), _full(b2f.shape)],
        out_specs=pl.BlockSpec((_B, _R, dim), lambda i: (i, 0, 0)),
        out_shape=jax.ShapeDtypeStruct((n // _R, _R, dim), jnp.float32),
    )(x3, W0f, b0f, W1f, b1f, W2f, b2f)
    return out3.reshape(n, dim)


# speedup vs baseline: 1.0287x; 1.0287x over previous
import jax
import jax.numpy as jnp
from jax.experimental import pallas as pl

_EPS = 1e-5
_R = 8
_B = 2500


def _accum(o_ref, part):
    @pl.when(pl.program_id(0) == 0)
    def _():
        o_ref[...] = part

    @pl.when(pl.program_id(0) != 0)
    def _():
        o_ref[...] += part


def _pack(xb, d):
    m = xb.shape[0]
    L = m // 4
    xp = jnp.pad(xb, ((0, 0), (0, 32 - d))) if d != 32 else xb
    return jnp.concatenate([xp[0:L], xp[L:2 * L], xp[2 * L:3 * L], xp[3 * L:4 * L]],
                           axis=1)


def _colstats(h):
    s = jnp.sum(h, axis=0, keepdims=True)
    sq = jnp.sum(h * h, axis=0, keepdims=True)
    return jnp.concatenate([s, sq], axis=0)


def _p1_kernel(x_ref, o_ref):
    b, r, d = x_ref.shape
    xb = x_ref[...].reshape(b * r, d)
    _accum(o_ref, _colstats(_pack(xb, d)))


def _p2_kernel(x_ref, w0_ref, b0_ref, o_ref):
    b, r, d = x_ref.shape
    xq = _pack(x_ref[...].reshape(b * r, d), d)
    h = jnp.dot(xq, w0_ref[...], preferred_element_type=jnp.float32)
    h = jnp.maximum(h + b0_ref[...], 0.0)
    _accum(o_ref, _colstats(h))


def _p3_kernel(x_ref, w0_ref, b0_ref, w1_ref, b1_ref, o_ref):
    b, r, d = x_ref.shape
    xq = _pack(x_ref[...].reshape(b * r, d), d)
    h = jnp.dot(xq, w0_ref[...], preferred_element_type=jnp.float32)
    h = jnp.maximum(h + b0_ref[...], 0.0)
    h = jnp.dot(h, w1_ref[...], preferred_element_type=jnp.float32)
    h = jnp.maximum(h + b1_ref[...], 0.0)
    _accum(o_ref, _colstats(h))


def _p4_kernel(x_ref, w0_ref, b0_ref, w1_ref, b1_ref, w2_ref, b2_ref, o_ref):
    b, r, d = x_ref.shape
    xq = _pack(x_ref[...].reshape(b * r, d), d)
    h = jnp.dot(xq, w0_ref[...], preferred_element_type=jnp.float32)
    h = jnp.maximum(h + b0_ref[...], 0.0)
    h = jnp.dot(h, w1_ref[...], preferred_element_type=jnp.float32)
    h = jnp.maximum(h + b1_ref[...], 0.0)
    h = jnp.dot(h, w2_ref[...], preferred_element_type=jnp.float32)
    h = jnp.maximum(h + b2_ref[...], 0.0)
    out = jnp.concatenate([h[:, 0:32], h[:, 32:64], h[:, 64:96], h[:, 96:128]],
                          axis=0)
    o_ref[...] = out.reshape(b, r, 32)


def _fold_packed(stats_packed, n, gamma, beta, W, b, groups):
    d = W.shape[0]
    stats = jnp.sum(stats_packed.reshape(2, groups, 32)[:, :, :d], axis=1)
    mean = stats[0] / n
    var = stats[1] / n - mean * mean
    inv = gamma * jax.lax.rsqrt(var + _EPS)
    Wf = W * inv[:, None]
    bf = b + (beta - mean * inv) @ W
    Wp = jnp.zeros((32, 32), jnp.float32).at[:d, :].set(Wf) if d != 32 else Wf
    bp = jnp.zeros((32,), jnp.float32) + bf
    Wbig = jnp.kron(jnp.eye(4, dtype=jnp.float32), Wp)
    bbig = jnp.tile(bp, 4)[None, :]
    return Wbig, bbig


def _full(shape):
    return pl.BlockSpec(shape, lambda i: tuple(0 for _ in shape))


def kernel(x, bn_g0, bn_b0, W0, b0, bn_g1, bn_b1, W1, b1, bn_g2, bn_b2,
           W2, b2):
    n, d_in = x.shape
    dim = W0.shape[1]
    x3 = x.reshape(n // _R, _R, d_in)
    grid = (n // (_R * _B),)
    x_spec = pl.BlockSpec((_B, _R, d_in), lambda i: (i, 0, 0))

    stats0 = pl.pallas_call(
        _p1_kernel,
        grid=grid,
        in_specs=[x_spec],
        out_specs=_full((2, 128)),
        out_shape=jax.ShapeDtypeStruct((2, 128), jnp.float32),
    )(x3)
    W0f, b0f = _fold_packed(stats0, n, bn_g0, bn_b0, W0, b0, 4)

    stats1 = pl.pallas_call(
        _p2_kernel,
        grid=grid,
        in_specs=[x_spec, _full(W0f.shape), _full(b0f.shape)],
        out_specs=_full((2, 128)),
        out_shape=jax.ShapeDtypeStruct((2, 128), jnp.float32),
    )(x3, W0f, b0f)
    W1f, b1f = _fold_packed(stats1, n, bn_g1, bn_b1, W1, b1, 4)

    stats2 = pl.pallas_call(
        _p3_kernel,
        grid=grid,
        in_specs=[x_spec, _full(W0f.shape), _full(b0f.shape),
                  _full(W1f.shape), _full(b1f.shape)],
        out_specs=_full((2, 128)),
        out_shape=jax.ShapeDtypeStruct((2, 128), jnp.float32),
    )(x3, W0f, b0f, W1f, b1f)
    W2f, b2f = _fold_packed(stats2, n, bn_g2, bn_b2, W2, b2, 4)

    out3 = pl.pallas_call(
        _p4_kernel,
        grid=grid,
        in_specs=[x_spec, _full(W0f.shape), _full(b0f.shape),
                  _full(W1f.shape), _full(b1f.shape),
                  _full(W2f.shape), _full(b2f.shape)],
        out_specs=pl.BlockSpec((_B, _R, dim), lambda i: (i, 0, 0)),
        out_shape=jax.ShapeDtypeStruct((n // _R, _R, dim), jnp.float32),
    )(x3, W0f, b0f, W1f, b1f, W2f, b2f)
    return out3.reshape(n, dim)


# speedup vs baseline: 1.0296x; 1.0009x over previous
import jax
import jax.numpy as jnp
from jax.experimental import pallas as pl

_EPS = 1e-5
_R = 8
_B = 2500


def _accum(o_ref, part):
    @pl.when(pl.program_id(0) == 0)
    def _():
        o_ref[...] = part

    @pl.when(pl.program_id(0) != 0)
    def _():
        o_ref[...] += part


def _pack(xb, d):
    m = xb.shape[0]
    L = m // 4
    xp = jnp.pad(xb, ((0, 0), (0, 32 - d))) if d != 32 else xb
    return jnp.concatenate([xp[0:L], xp[L:2 * L], xp[2 * L:3 * L], xp[3 * L:4 * L]],
                           axis=1)


def _colstats(h):
    s = jnp.sum(h, axis=0, keepdims=True)
    sq = jnp.sum(h * h, axis=0, keepdims=True)
    return jnp.concatenate([s, sq], axis=0)


def _p1_kernel(x_ref, o_ref):
    b, r, d = x_ref.shape
    xb = x_ref[...].reshape(b * r, d)
    _accum(o_ref, _colstats(_pack(xb, d)))


def _p2_kernel(x_ref, w0_ref, b0_ref, o_ref):
    b, r, d = x_ref.shape
    xq = _pack(x_ref[...].reshape(b * r, d), d).astype(jnp.bfloat16)
    h = jnp.dot(xq, w0_ref[...], preferred_element_type=jnp.float32)
    h = jnp.maximum(h + b0_ref[...], 0.0)
    _accum(o_ref, _colstats(h))


def _p3_kernel(x_ref, w0_ref, b0_ref, w1_ref, b1_ref, o_ref):
    b, r, d = x_ref.shape
    xq = _pack(x_ref[...].reshape(b * r, d), d).astype(jnp.bfloat16)
    h = jnp.dot(xq, w0_ref[...], preferred_element_type=jnp.float32)
    h = jnp.maximum(h + b0_ref[...], 0.0)
    h = jnp.dot(h.astype(jnp.bfloat16), w1_ref[...], preferred_element_type=jnp.float32)
    h = jnp.maximum(h + b1_ref[...], 0.0)
    _accum(o_ref, _colstats(h))


def _p4_kernel(x_ref, w0_ref, b0_ref, w1_ref, b1_ref, w2_ref, b2_ref, o_ref):
    b, r, d = x_ref.shape
    xq = _pack(x_ref[...].reshape(b * r, d), d).astype(jnp.bfloat16)
    h = jnp.dot(xq, w0_ref[...], preferred_element_type=jnp.float32)
    h = jnp.maximum(h + b0_ref[...], 0.0)
    h = jnp.dot(h.astype(jnp.bfloat16), w1_ref[...], preferred_element_type=jnp.float32)
    h = jnp.maximum(h + b1_ref[...], 0.0)
    h = jnp.dot(h.astype(jnp.bfloat16), w2_ref[...], preferred_element_type=jnp.float32)
    h = jnp.maximum(h + b2_ref[...], 0.0)
    out = jnp.concatenate([h[:, 0:32], h[:, 32:64], h[:, 64:96], h[:, 96:128]],
                          axis=0)
    o_ref[...] = out.reshape(b, r, 32)


def _fold_packed(stats_packed, n, gamma, beta, W, b, groups):
    d = W.shape[0]
    stats = jnp.sum(stats_packed.reshape(2, groups, 32)[:, :, :d], axis=1)
    mean = stats[0] / n
    var = stats[1] / n - mean * mean
    inv = gamma * jax.lax.rsqrt(var + _EPS)
    Wf = W * inv[:, None]
    bf = b + (beta - mean * inv) @ W
    Wp = jnp.zeros((32, 32), jnp.float32).at[:d, :].set(Wf) if d != 32 else Wf
    bp = jnp.zeros((32,), jnp.float32) + bf
    Wbig = jnp.kron(jnp.eye(4, dtype=jnp.float32), Wp).astype(jnp.bfloat16)
    bbig = jnp.tile(bp, 4)[None, :]
    return Wbig, bbig


def _full(shape):
    return pl.BlockSpec(shape, lambda i: tuple(0 for _ in shape))


def kernel(x, bn_g0, bn_b0, W0, b0, bn_g1, bn_b1, W1, b1, bn_g2, bn_b2,
           W2, b2):
    n, d_in = x.shape
    dim = W0.shape[1]
    x3 = x.reshape(n // _R, _R, d_in)
    grid = (n // (_R * _B),)
    x_spec = pl.BlockSpec((_B, _R, d_in), lambda i: (i, 0, 0))

    stats0 = pl.pallas_call(
        _p1_kernel,
        grid=grid,
        in_specs=[x_spec],
        out_specs=_full((2, 128)),
        out_shape=jax.ShapeDtypeStruct((2, 128), jnp.float32),
    )(x3)
    W0f, b0f = _fold_packed(stats0, n, bn_g0, bn_b0, W0, b0, 4)

    stats1 = pl.pallas_call(
        _p2_kernel,
        grid=grid,
        in_specs=[x_spec, _full(W0f.shape), _full(b0f.shape)],
        out_specs=_full((2, 128)),
        out_shape=jax.ShapeDtypeStruct((2, 128), jnp.float32),
    )(x3, W0f, b0f)
    W1f, b1f = _fold_packed(stats1, n, bn_g1, bn_b1, W1, b1, 4)

    stats2 = pl.pallas_call(
        _p3_kernel,
        grid=grid,
        in_specs=[x_spec, _full(W0f.shape), _full(b0f.shape),
                  _full(W1f.shape), _full(b1f.shape)],
        out_specs=_full((2, 128)),
        out_shape=jax.ShapeDtypeStruct((2, 128), jnp.float32),
    )(x3, W0f, b0f, W1f, b1f)
    W2f, b2f = _fold_packed(stats2, n, bn_g2, bn_b2, W2, b2, 4)

    out3 = pl.pallas_call(
        _p4_kernel,
        grid=grid,
        in_specs=[x_spec, _full(W0f.shape), _full(b0f.shape),
                  _full(W1f.shape), _full(b1f.shape),
                  _full(W2f.shape), _full(b2f.shape)],
        out_specs=pl.BlockSpec((_B, _R, dim), lambda i: (i, 0, 0)),
        out_shape=jax.ShapeDtypeStruct((n // _R, _R, dim), jnp.float32),
    )(x3, W0f, b0f, W1f, b1f, W2f, b2f)
    return out3.reshape(n, dim)
